# baseline (speedup 1.0000x reference)
import functools

import jax
import jax.numpy as jnp
from jax import lax
from jax.experimental import pallas as pl
from jax.experimental.pallas import tpu as pltpu
from jax.experimental.pallas import tpu_sc as plsc

B, S, H, D = 8, 2048, 8, 128
S_CHUNKS = 2
CS = S // S_CHUNKS


def _copy_body(ck, cv, ok, ov):
    ok[...] = ck[...]
    ov[...] = cv[...]


def _tc_copy(cached_key, cached_value):
    out_sds = jax.ShapeDtypeStruct((B, S, H, D), jnp.float32)
    cache_spec = pl.BlockSpec((1, CS, H, D), lambda c, b: (b, c, 0, 0))
    return pl.pallas_call(
        _copy_body,
        grid=(S_CHUNKS, B),
        in_specs=[cache_spec, cache_spec],
        out_specs=[cache_spec, cache_spec],
        out_shape=[out_sds, out_sds],
        compiler_params=pltpu.CompilerParams(
            dimension_semantics=("parallel", "parallel"),
        ),
    )(cached_key, cached_value)


_SC_MESH = plsc.VectorSubcoreMesh(core_axis_name="c", subcore_axis_name="s")


@functools.partial(
    pl.kernel,
    out_type=(),
    mesh=_SC_MESH,
    scratch_types=[
        pltpu.VMEM((B,), jnp.int32),
        pltpu.VMEM((B, H * D), jnp.float32),
        pltpu.SemaphoreType.DMA,
    ],
)
def _sc_scatter(kr, vr, kt, vt, idx_hbm, idx_v, rows_v, sem):
    cid = lax.axis_index("c")
    sid = lax.axis_index("s")
    wid = sid * 2 + cid

    @pl.when(wid == 0)
    def _():
        pltpu.sync_copy(idx_hbm, idx_v)
        pltpu.sync_copy(kt, rows_v)
        pltpu.async_copy(rows_v, kr.at[idx_v], sem).wait()

    @pl.when(wid == 1)
    def _():
        pltpu.sync_copy(idx_hbm, idx_v)
        pltpu.sync_copy(vt, rows_v)
        pltpu.async_copy(rows_v, vr.at[idx_v], sem).wait()


def kernel(cached_key, cached_value, key_token, value_token, lengths):
    ck2, cv2 = _tc_copy(cached_key, cached_value)
    kr = jax.new_ref(ck2.reshape(B * S, H * D))
    vr = jax.new_ref(cv2.reshape(B * S, H * D))
    idx = jnp.arange(B, dtype=jnp.int32) * S + lengths
    _sc_scatter(
        kr, vr,
        key_token.reshape(B, H * D),
        value_token.reshape(B, H * D),
        idx,
    )
    new_key = kr[...].reshape(B, S, H, D)
    new_value = vr[...].reshape(B, S, H, D)
    return (new_key, new_value)


# speedup vs baseline: 1.3948x; 1.3948x over previous
import functools

import jax
import jax.numpy as jnp
from jax import lax
from jax.experimental import pallas as pl
from jax.experimental.pallas import tpu as pltpu
from jax.experimental.pallas import tpu_sc as plsc

B, S, H, D = 8, 2048, 8, 128
ROWS = B * S
NW = 32
RPW = ROWS // NW
CHUNK = 32
NCHUNK = RPW // CHUNK


def _tc_body(len_ref, ck, kt, ok):
    b = pl.program_id(0)
    ok[...] = ck[...]
    l = len_ref[b]
    ok[0, pl.ds(l, 1)] = kt[pl.ds(b, 1), 0]


def _tc_key(cached_key, key_token, lengths):
    out_sds = jax.ShapeDtypeStruct((B, S, H, D), jnp.float32)
    cache_spec = pl.BlockSpec((1, S, H, D), lambda b: (b, 0, 0, 0))
    token_spec = pl.BlockSpec((B, 1, H, D), lambda b: (0, 0, 0, 0))
    return pl.pallas_call(
        _tc_body,
        grid=(B,),
        in_specs=[
            pl.BlockSpec(memory_space=pltpu.SMEM),
            cache_spec,
            token_spec,
        ],
        out_specs=cache_spec,
        out_shape=out_sds,
        compiler_params=pltpu.CompilerParams(
            dimension_semantics=("parallel",),
            vmem_limit_bytes=60 * 1024 * 1024,
        ),
    )(lengths, cached_key, key_token)


_SC_MESH = plsc.VectorSubcoreMesh(core_axis_name="c", subcore_axis_name="s")


@functools.partial(
    pl.kernel,
    out_type=jax.ShapeDtypeStruct((ROWS, H * D), jnp.float32),
    mesh=_SC_MESH,
    compiler_params=pltpu.CompilerParams(needs_layout_passes=False),
    scratch_types=[
        pltpu.VMEM((2, CHUNK, H * D), jnp.float32),
        pltpu.VMEM((16,), jnp.int32),
        pltpu.VMEM((B, H * D), jnp.float32),
        pltpu.SemaphoreType.DMA,
        pltpu.SemaphoreType.DMA,
    ],
)
def _sc_value(cv2d, vt2d, len16, out, ring, len_v, tok_v, sin, sout):
    cid = lax.axis_index("c")
    sid = lax.axis_index("s")
    wid = sid * 2 + cid
    base = wid * RPW

    pltpu.sync_copy(len16, len_v)
    pltpu.sync_copy(vt2d, tok_v)

    for i in range(NCHUNK):
        buf = i % 2
        if i >= 2:
            pltpu.make_async_copy(
                ring.at[buf], out.at[pl.ds(base + (i - 2) * CHUNK, CHUNK)], sout
            ).wait()
        cp_in = pltpu.make_async_copy(
            cv2d.at[pl.ds(base + i * CHUNK, CHUNK)], ring.at[buf], sin
        )
        cp_in.start()
        cp_in.wait()
        pltpu.make_async_copy(
            ring.at[buf], out.at[pl.ds(base + i * CHUNK, CHUNK)], sout
        ).start()
    for i in range(NCHUNK - 2, NCHUNK):
        pltpu.make_async_copy(
            ring.at[i % 2], out.at[pl.ds(base + i * CHUNK, CHUNK)], sout
        ).wait()

    lens = len_v[...]
    lane = lax.broadcasted_iota(jnp.int32, (16,), 0)
    for b in range(B):
        l_b = jnp.max(jnp.where(lane == b, lens, -1))
        row = b * S + l_b

        @pl.when((row >= base) & (row < base + RPW))
        def _(row=row, b=b):
            pltpu.sync_copy(tok_v.at[pl.ds(b, 1)], out.at[pl.ds(row, 1)])


def kernel(cached_key, cached_value, key_token, value_token, lengths):
    len16 = jnp.concatenate([lengths, jnp.zeros((8,), jnp.int32)])
    new_value = _sc_value(
        cached_value.reshape(ROWS, H * D),
        value_token.reshape(B, H * D),
        len16,
    )
    new_key = _tc_key(cached_key, key_token, lengths)
    return (new_key, new_value.reshape(B, S, H, D))


# speedup vs baseline: 1.4160x; 1.0152x over previous
import functools

import jax
import jax.numpy as jnp
from jax import lax
from jax.experimental import pallas as pl
from jax.experimental.pallas import tpu as pltpu
from jax.experimental.pallas import tpu_sc as plsc

B, S, H, D = 8, 2048, 8, 128
ROWS = B * S
NW = 32
RPW = ROWS // NW
CHUNK = 16
NCHUNK = RPW // CHUNK
NBUF = 6


def _tc_body(len_ref, ck, kt, ok):
    b = pl.program_id(0)
    ok[...] = ck[...]
    l = len_ref[b]
    ok[0, pl.ds(l, 1)] = kt[pl.ds(b, 1), 0]


def _tc_key(cached_key, key_token, lengths):
    out_sds = jax.ShapeDtypeStruct((B, S, H, D), jnp.float32)
    cache_spec = pl.BlockSpec((1, S, H, D), lambda b: (b, 0, 0, 0))
    token_spec = pl.BlockSpec((B, 1, H, D), lambda b: (0, 0, 0, 0))
    return pl.pallas_call(
        _tc_body,
        grid=(B,),
        in_specs=[
            pl.BlockSpec(memory_space=pltpu.SMEM),
            cache_spec,
            token_spec,
        ],
        out_specs=cache_spec,
        out_shape=out_sds,
        compiler_params=pltpu.CompilerParams(
            dimension_semantics=("parallel",),
            vmem_limit_bytes=60 * 1024 * 1024,
        ),
    )(lengths, cached_key, key_token)


_SC_MESH = plsc.VectorSubcoreMesh(core_axis_name="c", subcore_axis_name="s")


@functools.partial(
    pl.kernel,
    out_type=jax.ShapeDtypeStruct((ROWS, H * D), jnp.float32),
    mesh=_SC_MESH,
    compiler_params=pltpu.CompilerParams(needs_layout_passes=False),
    scratch_types=[
        pltpu.VMEM((NBUF, CHUNK, H * D), jnp.float32),
        pltpu.VMEM((16,), jnp.int32),
        pltpu.VMEM((B, H * D), jnp.float32),
        pltpu.SemaphoreType.DMA,
        pltpu.SemaphoreType.DMA,
    ],
)
def _sc_value(cv2d, vt2d, len16, out, ring, len_v, tok_v, sin, sout):
    cid = lax.axis_index("c")
    sid = lax.axis_index("s")
    wid = sid * 2 + cid
    base = wid * RPW

    pltpu.sync_copy(len16, len_v)
    pltpu.sync_copy(vt2d, tok_v)

    def _load(i):
        return pltpu.make_async_copy(
            cv2d.at[pl.ds(base + i * CHUNK, CHUNK)], ring.at[i % NBUF], sin
        )

    def _store(i):
        return pltpu.make_async_copy(
            ring.at[i % NBUF], out.at[pl.ds(base + i * CHUNK, CHUNK)], sout
        )

    _load(0).start()
    _load(1).start()
    for i in range(NCHUNK):
        if i >= 4:
            _store(i - 4).wait()
        if i + 2 < NCHUNK:
            _load(i + 2).start()
        _load(i).wait()
        _store(i).start()
    for i in range(NCHUNK - 4, NCHUNK):
        _store(i).wait()

    lens = len_v[...]
    lane = lax.broadcasted_iota(jnp.int32, (16,), 0)
    for b in range(B):
        l_b = jnp.max(jnp.where(lane == b, lens, -1))
        row = b * S + l_b

        @pl.when((row >= base) & (row < base + RPW))
        def _(row=row, b=b):
            pltpu.sync_copy(tok_v.at[pl.ds(b, 1)], out.at[pl.ds(row, 1)])


def kernel(cached_key, cached_value, key_token, value_token, lengths):
    len16 = jnp.concatenate([lengths, jnp.zeros((8,), jnp.int32)])
    new_value = _sc_value(
        cached_value.reshape(ROWS, H * D),
        value_token.reshape(B, H * D),
        len16,
    )
    new_key = _tc_key(cached_key, key_token, lengths)
    return (new_key, new_value.reshape(B, S, H, D))
